# speedup vs baseline: 1.1687x; 1.1687x over previous
import functools

import jax
import jax.numpy as jnp
from jax import lax
from jax.experimental import pallas as pl
from jax.experimental.pallas import tpu as pltpu
from jax.experimental.pallas import tpu_sc as plsc

N_NODES = 10000
DIM = 256
N_EDGES = 160000
NUM_LAYER = 3

NT = 32
ROWS = 320
NPAD = NT * ROWS
TRASH = ROWS
ACC_ROWS = ROWS + 8
CHUNK = 64
EPAD = N_EDGES + 2 * CHUNK

_mesh = plsc.VectorSubcoreMesh(core_axis_name="c", subcore_axis_name="s")


def _mo(v, m):
    return pl.multiple_of(v, m)


def _layer_body(x_hbm, srcs_hbm, dsts_hbm, nrm_hbm, starts_hbm, dis_hbm,
                out_hbm, src_v, dst_v, nrm_v, rows_v, acc_v, xbuf_v,
                dis_v, meta_v, sem):
    wid = lax.axis_index("c") * 16 + lax.axis_index("s")
    vbase = _mo(wid * ROWS, 8)

    pltpu.sync_copy(starts_hbm.at[pl.ds(_mo(wid * 16, 16), 16)], meta_v)
    mvec = meta_v[...]
    s0 = mvec[0]
    s1 = mvec[1]
    abase = _mo(s0 & ~7, 8)
    nchunks = (s1 - abase + CHUNK - 1) // CHUNK

    zeros16 = jnp.zeros((16,), jnp.float32)

    @pl.loop(0, ACC_ROWS)
    def _(r):
        for c in range(0, DIM, 16):
            acc_v[r, pl.ds(c, 16)] = zeros16

    @pl.loop(0, nchunks)
    def _(ci):
        eoff = _mo(abase + ci * CHUNK, 8)
        pltpu.sync_copy(srcs_hbm.at[pl.ds(eoff, CHUNK)], src_v)
        pltpu.sync_copy(dsts_hbm.at[pl.ds(eoff, CHUNK)], dst_v)
        pltpu.sync_copy(nrm_hbm.at[pl.ds(eoff, CHUNK)], nrm_v)

        for g in range(CHUNK // 16):
            pos = eoff + g * 16 + lax.iota(jnp.int32, 16)
            valid = (pos >= s0) & (pos < s1)
            d16 = dst_v[pl.ds(g * 16, 16)]
            dst_v[pl.ds(g * 16, 16)] = jnp.where(valid, d16 - vbase, TRASH)

        pltpu.async_copy(x_hbm.at[src_v], rows_v, sem).wait()

        for g in range(CHUNK // 16):
            dvec = dst_v[pl.ds(g * 16, 16)]
            nvec = nrm_v[pl.ds(g * 16, 16)]
            for j in range(16):
                dloc = dvec[j]
                s = nvec[j]
                e = g * 16 + j
                for c in range(0, DIM, 16):
                    acc_v[dloc, pl.ds(c, 16)] += s * rows_v[e, pl.ds(c, 16)]

    pltpu.sync_copy(dis_hbm.at[pl.ds(vbase, ROWS)], dis_v)

    @pl.loop(0, ROWS // 16)
    def _(rg):
        rb = _mo(rg * 16, 16)
        pltpu.sync_copy(x_hbm.at[pl.ds(vbase + rb, 16)], xbuf_v)
        dvals = dis_v[pl.ds(rb, 16)]
        for j in range(16):
            s = dvals[j]
            for c in range(0, DIM, 16):
                xbuf_v[j, pl.ds(c, 16)] = (
                    xbuf_v[j, pl.ds(c, 16)] + s * acc_v[rb + j, pl.ds(c, 16)]
                )
        pltpu.sync_copy(xbuf_v, out_hbm.at[pl.ds(vbase + rb, 16)])


_propagate = functools.partial(
    pl.kernel,
    out_type=jax.ShapeDtypeStruct((NPAD, DIM), jnp.float32),
    mesh=_mesh,
    scratch_types=[
        pltpu.VMEM((CHUNK,), jnp.int32),
        pltpu.VMEM((CHUNK,), jnp.int32),
        pltpu.VMEM((CHUNK,), jnp.float32),
        pltpu.VMEM((CHUNK, DIM), jnp.float32),
        pltpu.VMEM((ACC_ROWS, DIM), jnp.float32),
        pltpu.VMEM((16, DIM), jnp.float32),
        pltpu.VMEM((ROWS,), jnp.float32),
        pltpu.VMEM((16,), jnp.int32),
        pltpu.SemaphoreType.DMA,
    ],
)(_layer_body)


def kernel(edge_index, user, item):
    src = edge_index[0].astype(jnp.int32)
    dst = edge_index[1].astype(jnp.int32)
    x = jnp.concatenate([user, item], axis=0)

    mask_f = (src != dst).astype(jnp.float32)
    deg = jnp.zeros((N_NODES,), jnp.float32).at[src].add(mask_f)
    dis = jnp.where(deg > 0, lax.rsqrt(deg), 0.0)

    perm = jnp.argsort(dst)
    srcs_s = jnp.pad(src[perm], (0, EPAD - N_EDGES))
    dsts_s = jnp.pad(dst[perm], (0, EPAD - N_EDGES))
    nrm_s = jnp.pad((dis[src] * mask_f)[perm], (0, EPAD - N_EDGES))
    bounds = jnp.searchsorted(
        dsts_s[:N_EDGES], jnp.arange(NT + 1, dtype=jnp.int32) * ROWS
    ).astype(jnp.int32)
    starts = jnp.zeros((NT, 16), jnp.int32)
    starts = starts.at[:, 0].set(bounds[:NT]).at[:, 1].set(bounds[1:])
    starts = starts.reshape(-1)

    x_pad = jnp.pad(x, ((0, NPAD - N_NODES), (0, 0)))
    dis_pad = jnp.pad(dis, (0, NPAD - N_NODES))

    for _ in range(NUM_LAYER):
        x_pad = _propagate(x_pad, srcs_s, dsts_s, nrm_s, starts, dis_pad)
    return x_pad[:N_NODES]


# speedup vs baseline: 1.5228x; 1.3031x over previous
import dataclasses
import functools

import jax
import jax.numpy as jnp
from jax import lax
from jax.experimental import pallas as pl
from jax.experimental.pallas import tpu as pltpu
from jax.experimental.pallas import tpu_sc as plsc

N_NODES = 10000
DIM = 256
N_EDGES = 160000
NUM_LAYER = 3

NT = 32
ROWS = 320
NPAD = NT * ROWS
TRASH = ROWS
ACC_ROWS = ROWS + 1
CHUNK = 32
EPAD = N_EDGES + 8 * CHUNK
RG = ROWS // 16

_mesh = plsc.VectorSubcoreMesh(core_axis_name="c", subcore_axis_name="s")


def _mo(v, m):
    return pl.multiple_of(v, m)


def _layer_body(x_hbm, srcs_hbm, meta_hbm, starts_hbm, dis_hbm, out_hbm,
                sidx0, sidx1, m0, m1, rows0, rows1, acc_v, xb0, xb1,
                dis_v, meta_v, ms0a, ms0b, ms1a, ms1b, gs0, gs1,
                xi0, xi1, xo):
    wid = lax.axis_index("c") * 16 + lax.axis_index("s")
    vbase = _mo(wid * ROWS, 8)

    pltpu.sync_copy(starts_hbm.at[pl.ds(_mo(wid * 16, 16), 16)], meta_v)
    mvec = meta_v[...]
    s0 = mvec[0]
    s1 = mvec[1]
    abase = _mo(s0 & ~15, 16)
    npairs = (s1 - abase + 2 * CHUNK - 1) // (2 * CHUNK)

    zeros16 = jnp.zeros((16,), jnp.float32)

    @pl.loop(0, ACC_ROWS)
    def _(r):
        for c in range(0, DIM, 16):
            acc_v[r, pl.ds(c, 16)] = zeros16

    def issue_meta(ci, sidx, mb, sa, sb):
        eoff = _mo(abase + ci * CHUNK, 8)
        pltpu.async_copy(srcs_hbm.at[pl.ds(eoff, CHUNK)], sidx, sa)
        pltpu.async_copy(
            meta_hbm.at[pl.ds(_mo(2 * eoff, 16), 2 * CHUNK)], mb, sb)

    def wait_meta(sidx, mb, sa, sb):
        pltpu.make_async_copy(srcs_hbm.at[pl.ds(0, CHUNK)], sidx, sa).wait()
        pltpu.make_async_copy(
            meta_hbm.at[pl.ds(0, 2 * CHUNK)], mb, sb).wait()

    def accumulate(ci, mb, rows):
        eoff = abase + ci * CHUNK
        for g in range(CHUNK // 16):
            pos = eoff + g * 16 + lax.iota(jnp.int32, 16)
            valid = (pos >= s0) & (pos < s1)
            dvec = jnp.where(
                valid, mb[pl.ds(32 * g, 16)].astype(jnp.int32), TRASH)
            nvec = mb[pl.ds(32 * g + 16, 16)]
            dscal = [dvec[j] for j in range(16)]
            nscal = [nvec[j] for j in range(16)]

            @pl.loop(0, DIM, step=16)
            def _(c):
                cc = _mo(c, 16)
                for j in range(16):
                    e = g * 16 + j
                    acc_v[dscal[j], pl.ds(cc, 16)] += (
                        nscal[j] * rows[e, pl.ds(cc, 16)]
                    )

    issue_meta(0, sidx0, m0, ms0a, ms0b)
    issue_meta(1, sidx1, m1, ms1a, ms1b)

    @pl.loop(0, npairs)
    def _(p):
        ca = 2 * p
        cb = 2 * p + 1
        wait_meta(sidx0, m0, ms0a, ms0b)
        pltpu.async_copy(x_hbm.at[sidx0], rows0, gs0)
        wait_meta(sidx1, m1, ms1a, ms1b)
        pltpu.async_copy(x_hbm.at[sidx1], rows1, gs1)
        pltpu.make_async_copy(x_hbm.at[sidx0], rows0, gs0).wait()
        accumulate(ca, m0, rows0)
        issue_meta(ca + 2, sidx0, m0, ms0a, ms0b)
        pltpu.make_async_copy(x_hbm.at[sidx1], rows1, gs1).wait()
        accumulate(cb, m1, rows1)
        issue_meta(cb + 2, sidx1, m1, ms1a, ms1b)

    wait_meta(sidx0, m0, ms0a, ms0b)
    wait_meta(sidx1, m1, ms1a, ms1b)

    pltpu.sync_copy(dis_hbm.at[pl.ds(vbase, ROWS)], dis_v)

    def issue_xin(rg, xb, sem):
        pltpu.async_copy(
            x_hbm.at[pl.ds(vbase + _mo(rg * 16, 16), 16)], xb, sem)

    issue_xin(0, xb0, xi0)
    issue_xin(1, xb1, xi1)

    def epi_step(rg, xb, sem):
        rb = _mo(rg * 16, 16)
        pltpu.make_async_copy(x_hbm.at[pl.ds(vbase, 16)], xb, sem).wait()
        dvals = dis_v[pl.ds(rb, 16)]
        dscal = [dvals[j] for j in range(16)]

        @pl.loop(0, DIM, step=16)
        def _(c):
            cc = _mo(c, 16)
            for j in range(16):
                acc_v[rb + j, pl.ds(cc, 16)] = (
                    xb[j, pl.ds(cc, 16)]
                    + dscal[j] * acc_v[rb + j, pl.ds(cc, 16)]
                )
        pltpu.async_copy(acc_v.at[pl.ds(rb, 16)],
                         out_hbm.at[pl.ds(vbase + rb, 16)], xo)

        @pl.when(rg + 2 < RG)
        def _():
            issue_xin(rg + 2, xb, sem)

    @pl.loop(0, RG // 2)
    def _(q):
        epi_step(2 * q, xb0, xi0)
        epi_step(2 * q + 1, xb1, xi1)

    @pl.loop(0, RG)
    def _(r):
        pltpu.make_async_copy(acc_v.at[pl.ds(0, 16)],
                              out_hbm.at[pl.ds(0, 16)], xo).wait()


_propagate = functools.partial(
    pl.kernel,
    out_type=jax.ShapeDtypeStruct((NPAD, DIM), jnp.float32),
    mesh=_mesh,
    scratch_types=[
        pltpu.VMEM((CHUNK,), jnp.int32),
        pltpu.VMEM((CHUNK,), jnp.int32),
        pltpu.VMEM((2 * CHUNK,), jnp.float32),
        pltpu.VMEM((2 * CHUNK,), jnp.float32),
        pltpu.VMEM((CHUNK, DIM), jnp.float32),
        pltpu.VMEM((CHUNK, DIM), jnp.float32),
        pltpu.VMEM((ACC_ROWS, DIM), jnp.float32),
        pltpu.VMEM((16, DIM), jnp.float32),
        pltpu.VMEM((16, DIM), jnp.float32),
        pltpu.VMEM((ROWS,), jnp.float32),
        pltpu.VMEM((16,), jnp.int32),
        pltpu.SemaphoreType.DMA,
        pltpu.SemaphoreType.DMA,
        pltpu.SemaphoreType.DMA,
        pltpu.SemaphoreType.DMA,
        pltpu.SemaphoreType.DMA,
        pltpu.SemaphoreType.DMA,
        pltpu.SemaphoreType.DMA,
        pltpu.SemaphoreType.DMA,
        pltpu.SemaphoreType.DMA,
    ],
)(_layer_body)


def kernel(edge_index, user, item):
    src = edge_index[0].astype(jnp.int32)
    dst = edge_index[1].astype(jnp.int32)
    x = jnp.concatenate([user, item], axis=0)

    mask_f = (src != dst).astype(jnp.float32)
    deg = jnp.zeros((N_NODES,), jnp.float32).at[src].add(mask_f)
    dis = jnp.where(deg > 0, lax.rsqrt(deg), 0.0)

    perm = jnp.argsort(dst)
    srcs_s = jnp.pad(src[perm], (0, EPAD - N_EDGES))
    dsts_s = jnp.pad(dst[perm], (0, EPAD - N_EDGES))
    nrm_s = jnp.pad((dis[src] * mask_f)[perm], (0, EPAD - N_EDGES))
    meta = jnp.stack(
        [
            (dsts_s % ROWS).astype(jnp.float32).reshape(-1, 16),
            nrm_s.reshape(-1, 16),
        ],
        axis=1,
    ).reshape(-1)
    bounds = jnp.searchsorted(
        dsts_s[:N_EDGES], jnp.arange(NT + 1, dtype=jnp.int32) * ROWS
    ).astype(jnp.int32)
    starts = jnp.zeros((NT, 16), jnp.int32)
    starts = starts.at[:, 0].set(bounds[:NT]).at[:, 1].set(bounds[1:])
    starts = starts.reshape(-1)

    x_pad = jnp.pad(x, ((0, NPAD - N_NODES), (0, 0)))
    dis_pad = jnp.pad(dis, (0, NPAD - N_NODES))

    for _ in range(NUM_LAYER):
        x_pad = _propagate(x_pad, srcs_s, meta, starts, dis_pad)
    return x_pad[:N_NODES]


# speedup vs baseline: 1.7370x; 1.1406x over previous
import functools

import jax
import jax.numpy as jnp
from jax import lax
from jax.experimental import pallas as pl
from jax.experimental.pallas import tpu as pltpu
from jax.experimental.pallas import tpu_sc as plsc

N_NODES = 10000
DIM = 256
N_EDGES = 160000
NUM_LAYER = 3

NT = 32
ROWS = 320
NPAD = NT * ROWS
TRASH = ROWS
ACC_ROWS = ROWS + 1
CHUNK = 32
NBUF = 4
EPAD = N_EDGES + 16 * CHUNK
RG = ROWS // 16

_mesh = plsc.VectorSubcoreMesh(core_axis_name="c", subcore_axis_name="s")


def _mo(v, m):
    return pl.multiple_of(v, m)


def _layer_body(x_hbm, srcs_hbm, meta_hbm, starts_hbm, dis_hbm, out_hbm,
                *scratch):
    sidx = scratch[0:NBUF]
    mb = scratch[NBUF:2 * NBUF]
    rows = scratch[2 * NBUF:3 * NBUF]
    acc_v, xb0, xb1, dis_v, meta_v = scratch[3 * NBUF:3 * NBUF + 5]
    sems = scratch[3 * NBUF + 5:]
    msa = sems[0:NBUF]
    msb = sems[NBUF:2 * NBUF]
    gs = sems[2 * NBUF:3 * NBUF]
    xi0, xi1, xo = sems[3 * NBUF:]

    wid = lax.axis_index("c") * 16 + lax.axis_index("s")
    vbase = _mo(wid * ROWS, 8)

    pltpu.sync_copy(starts_hbm.at[pl.ds(_mo(wid * 16, 16), 16)], meta_v)
    mvec = meta_v[...]
    s0 = mvec[0]
    s1 = mvec[1]
    abase = _mo(s0 & ~15, 16)
    nq = (s1 - abase + NBUF * CHUNK - 1) // (NBUF * CHUNK)

    def issue_meta(ci, k):
        eoff = _mo(abase + ci * CHUNK, 16)
        pltpu.async_copy(srcs_hbm.at[pl.ds(eoff, CHUNK)], sidx[k], msa[k])
        pltpu.async_copy(
            meta_hbm.at[pl.ds(_mo(2 * eoff, 16), 2 * CHUNK)], mb[k], msb[k])

    def wait_meta(k):
        pltpu.make_async_copy(
            srcs_hbm.at[pl.ds(0, CHUNK)], sidx[k], msa[k]).wait()
        pltpu.make_async_copy(
            meta_hbm.at[pl.ds(0, 2 * CHUNK)], mb[k], msb[k]).wait()

    def accumulate(ci, k):
        eoff = abase + ci * CHUNK
        for g in range(CHUNK // 16):
            pos = eoff + g * 16 + lax.iota(jnp.int32, 16)
            valid = (pos >= s0) & (pos < s1)
            dvec = jnp.where(
                valid, mb[k][pl.ds(32 * g, 16)].astype(jnp.int32), TRASH)
            nvec = mb[k][pl.ds(32 * g + 16, 16)]
            dscal = [dvec[j] for j in range(16)]
            nscal = [nvec[j] for j in range(16)]

            @pl.loop(0, DIM, step=16)
            def _(c):
                cc = _mo(c, 16)
                for j in range(16):
                    e = g * 16 + j
                    acc_v[dscal[j], pl.ds(cc, 16)] += (
                        nscal[j] * rows[k][e, pl.ds(cc, 16)]
                    )

    for k in range(NBUF):
        issue_meta(k, k)

    zeros16 = jnp.zeros((16,), jnp.float32)

    @pl.loop(0, ACC_ROWS)
    def _(r):
        for c in range(0, DIM, 16):
            acc_v[r, pl.ds(c, 16)] = zeros16

    @pl.loop(0, nq)
    def _(q):
        base = NBUF * q
        for k in range(NBUF):
            wait_meta(k)
            pltpu.async_copy(x_hbm.at[sidx[k]], rows[k], gs[k])
        for k in range(NBUF):
            pltpu.make_async_copy(x_hbm.at[sidx[k]], rows[k], gs[k]).wait()
            accumulate(base + k, k)
            issue_meta(base + k + NBUF, k)

    for k in range(NBUF):
        wait_meta(k)

    pltpu.sync_copy(dis_hbm.at[pl.ds(vbase, ROWS)], dis_v)

    def issue_xin(rg, xb, sem):
        pltpu.async_copy(
            x_hbm.at[pl.ds(vbase + _mo(rg * 16, 16), 16)], xb, sem)

    issue_xin(0, xb0, xi0)
    issue_xin(1, xb1, xi1)

    def epi_step(rg, xb, sem):
        rb = _mo(rg * 16, 16)
        pltpu.make_async_copy(x_hbm.at[pl.ds(vbase, 16)], xb, sem).wait()
        dvals = dis_v[pl.ds(rb, 16)]
        dscal = [dvals[j] for j in range(16)]

        @pl.loop(0, DIM, step=16)
        def _(c):
            cc = _mo(c, 16)
            for j in range(16):
                acc_v[rb + j, pl.ds(cc, 16)] = (
                    xb[j, pl.ds(cc, 16)]
                    + dscal[j] * acc_v[rb + j, pl.ds(cc, 16)]
                )
        pltpu.async_copy(acc_v.at[pl.ds(rb, 16)],
                         out_hbm.at[pl.ds(vbase + rb, 16)], xo)

        @pl.when(rg + 2 < RG)
        def _():
            issue_xin(rg + 2, xb, sem)

    @pl.loop(0, RG // 2)
    def _(q):
        epi_step(2 * q, xb0, xi0)
        epi_step(2 * q + 1, xb1, xi1)

    @pl.loop(0, RG)
    def _(r):
        pltpu.make_async_copy(acc_v.at[pl.ds(0, 16)],
                              out_hbm.at[pl.ds(0, 16)], xo).wait()


_propagate = functools.partial(
    pl.kernel,
    out_type=jax.ShapeDtypeStruct((NPAD, DIM), jnp.float32),
    mesh=_mesh,
    scratch_types=(
        [pltpu.VMEM((CHUNK,), jnp.int32) for _ in range(NBUF)]
        + [pltpu.VMEM((2 * CHUNK,), jnp.float32) for _ in range(NBUF)]
        + [pltpu.VMEM((CHUNK, DIM), jnp.float32) for _ in range(NBUF)]
        + [
            pltpu.VMEM((ACC_ROWS, DIM), jnp.float32),
            pltpu.VMEM((16, DIM), jnp.float32),
            pltpu.VMEM((16, DIM), jnp.float32),
            pltpu.VMEM((ROWS,), jnp.float32),
            pltpu.VMEM((16,), jnp.int32),
        ]
        + [pltpu.SemaphoreType.DMA for _ in range(3 * NBUF + 3)]
    ),
)(_layer_body)


def kernel(edge_index, user, item):
    src = edge_index[0].astype(jnp.int32)
    dst = edge_index[1].astype(jnp.int32)
    x = jnp.concatenate([user, item], axis=0)

    mask_f = (src != dst).astype(jnp.float32)
    deg = jnp.zeros((N_NODES,), jnp.float32).at[src].add(mask_f)
    dis = jnp.where(deg > 0, lax.rsqrt(deg), 0.0)

    dst_s, src_s, nrm_raw = lax.sort(
        (dst, src, dis[src] * mask_f), num_keys=1, is_stable=False)
    srcs_s = jnp.pad(src_s, (0, EPAD - N_EDGES))
    dsts_s = jnp.pad(dst_s, (0, EPAD - N_EDGES))
    nrm_s = jnp.pad(nrm_raw, (0, EPAD - N_EDGES))
    meta = jnp.stack(
        [
            (dsts_s % ROWS).astype(jnp.float32).reshape(-1, 16),
            nrm_s.reshape(-1, 16),
        ],
        axis=1,
    ).reshape(-1)
    bounds = jnp.searchsorted(
        dst_s, jnp.arange(NT + 1, dtype=jnp.int32) * ROWS
    ).astype(jnp.int32)
    starts = jnp.zeros((NT, 16), jnp.int32)
    starts = starts.at[:, 0].set(bounds[:NT]).at[:, 1].set(bounds[1:])
    starts = starts.reshape(-1)

    x_pad = jnp.pad(x, ((0, NPAD - N_NODES), (0, 0)))
    dis_pad = jnp.pad(dis, (0, NPAD - N_NODES))

    for _ in range(NUM_LAYER):
        x_pad = _propagate(x_pad, srcs_s, meta, starts, dis_pad)
    return x_pad[:N_NODES]


# speedup vs baseline: 1.9178x; 1.1041x over previous
import functools

import jax
import jax.numpy as jnp
from jax import lax
from jax.experimental import pallas as pl
from jax.experimental.pallas import tpu as pltpu
from jax.experimental.pallas import tpu_sc as plsc

N_NODES = 10000
DIM = 256
N_EDGES = 160000
NUM_LAYER = 3

NT = 32
ROWS = 320
NPAD = NT * ROWS
TRASH = ROWS
ACC_ROWS = ROWS + 1
CHUNK = 32
NBUF = 4
EPAD = N_EDGES + 16 * CHUNK
RG = ROWS // 16

_mesh = plsc.VectorSubcoreMesh(core_axis_name="c", subcore_axis_name="s")


def _mo(v, m):
    return pl.multiple_of(v, m)


def _layer_body(x_hbm, srcs_hbm, meta_hbm, starts_hbm, dis_hbm, out_hbm,
                *scratch):
    sidx = scratch[0:NBUF]
    mb = scratch[NBUF:2 * NBUF]
    rows = scratch[2 * NBUF:3 * NBUF]
    acc_v, xb0, xb1, dis_v, meta_v = scratch[3 * NBUF:3 * NBUF + 5]
    sems = scratch[3 * NBUF + 5:]
    msa = sems[0:NBUF]
    msb = sems[NBUF:2 * NBUF]
    gs = sems[2 * NBUF:3 * NBUF]
    xi0, xi1, xo = sems[3 * NBUF:]

    wid = lax.axis_index("c") * 16 + lax.axis_index("s")
    vbase = _mo(wid * ROWS, 8)

    pltpu.sync_copy(starts_hbm.at[pl.ds(_mo(wid * 16, 16), 16)], meta_v)
    mvec = meta_v[...]
    s0 = mvec[0]
    s1 = mvec[1]
    abase = _mo(s0 & ~15, 16)
    nq = (s1 - abase + NBUF * CHUNK - 1) // (NBUF * CHUNK)

    def issue_meta(ci, k):
        eoff = _mo(abase + ci * CHUNK, 16)
        pltpu.async_copy(srcs_hbm.at[pl.ds(eoff, CHUNK)], sidx[k], msa[k])
        pltpu.async_copy(
            meta_hbm.at[pl.ds(_mo(2 * eoff, 16), 2 * CHUNK)], mb[k], msb[k])

    def wait_meta(k):
        pltpu.make_async_copy(
            srcs_hbm.at[pl.ds(0, CHUNK)], sidx[k], msa[k]).wait()
        pltpu.make_async_copy(
            meta_hbm.at[pl.ds(0, 2 * CHUNK)], mb[k], msb[k]).wait()

    def accumulate(ci, k):
        eoff = abase + ci * CHUNK
        for g in range(CHUNK // 16):
            pos = eoff + g * 16 + lax.iota(jnp.int32, 16)
            valid = (pos >= s0) & (pos < s1)
            dvec = jnp.where(
                valid, mb[k][pl.ds(32 * g, 16)].astype(jnp.int32), TRASH)
            nvec = mb[k][pl.ds(32 * g + 16, 16)]
            dscal = [dvec[j] for j in range(16)]
            nscal = [nvec[j] for j in range(16)]

            @pl.loop(0, DIM, step=16)
            def _(c):
                cc = _mo(c, 16)
                for j in range(16):
                    e = g * 16 + j
                    plsc.addupdate(
                        acc_v.at[dscal[j], pl.ds(cc, 16)],
                        nscal[j] * rows[k][e, pl.ds(cc, 16)],
                    )

    for k in range(NBUF):
        issue_meta(k, k)

    zeros16 = jnp.zeros((16,), jnp.float32)

    @pl.loop(0, ACC_ROWS)
    def _(r):
        for c in range(0, DIM, 16):
            acc_v[r, pl.ds(c, 16)] = zeros16

    @pl.loop(0, nq)
    def _(q):
        base = NBUF * q
        for k in range(NBUF):
            wait_meta(k)
            pltpu.async_copy(x_hbm.at[sidx[k]], rows[k], gs[k])
        for k in range(NBUF):
            pltpu.make_async_copy(x_hbm.at[sidx[k]], rows[k], gs[k]).wait()
            accumulate(base + k, k)
            issue_meta(base + k + NBUF, k)

    for k in range(NBUF):
        wait_meta(k)

    pltpu.sync_copy(dis_hbm.at[pl.ds(vbase, ROWS)], dis_v)

    def issue_xin(rg, xb, sem):
        pltpu.async_copy(
            x_hbm.at[pl.ds(vbase + _mo(rg * 16, 16), 16)], xb, sem)

    issue_xin(0, xb0, xi0)
    issue_xin(1, xb1, xi1)

    def epi_step(rg, xb, sem):
        rb = _mo(rg * 16, 16)
        pltpu.make_async_copy(x_hbm.at[pl.ds(vbase, 16)], xb, sem).wait()
        dvals = dis_v[pl.ds(rb, 16)]
        dscal = [dvals[j] for j in range(16)]

        @pl.loop(0, DIM, step=16)
        def _(c):
            cc = _mo(c, 16)
            for j in range(16):
                acc_v[rb + j, pl.ds(cc, 16)] = (
                    xb[j, pl.ds(cc, 16)]
                    + dscal[j] * acc_v[rb + j, pl.ds(cc, 16)]
                )
        pltpu.async_copy(acc_v.at[pl.ds(rb, 16)],
                         out_hbm.at[pl.ds(vbase + rb, 16)], xo)

        @pl.when(rg + 2 < RG)
        def _():
            issue_xin(rg + 2, xb, sem)

    @pl.loop(0, RG // 2)
    def _(q):
        epi_step(2 * q, xb0, xi0)
        epi_step(2 * q + 1, xb1, xi1)

    @pl.loop(0, RG)
    def _(r):
        pltpu.make_async_copy(acc_v.at[pl.ds(0, 16)],
                              out_hbm.at[pl.ds(0, 16)], xo).wait()


_propagate = functools.partial(
    pl.kernel,
    out_type=jax.ShapeDtypeStruct((NPAD, DIM), jnp.float32),
    mesh=_mesh,
    scratch_types=(
        [pltpu.VMEM((CHUNK,), jnp.int32) for _ in range(NBUF)]
        + [pltpu.VMEM((2 * CHUNK,), jnp.float32) for _ in range(NBUF)]
        + [pltpu.VMEM((CHUNK, DIM), jnp.float32) for _ in range(NBUF)]
        + [
            pltpu.VMEM((ACC_ROWS, DIM), jnp.float32),
            pltpu.VMEM((16, DIM), jnp.float32),
            pltpu.VMEM((16, DIM), jnp.float32),
            pltpu.VMEM((ROWS,), jnp.float32),
            pltpu.VMEM((16,), jnp.int32),
        ]
        + [pltpu.SemaphoreType.DMA for _ in range(3 * NBUF + 3)]
    ),
)(_layer_body)


def kernel(edge_index, user, item):
    src = edge_index[0].astype(jnp.int32)
    dst = edge_index[1].astype(jnp.int32)
    x = jnp.concatenate([user, item], axis=0)

    mask_f = (src != dst).astype(jnp.float32)
    deg = jnp.zeros((N_NODES,), jnp.float32).at[src].add(mask_f)
    dis = jnp.where(deg > 0, lax.rsqrt(deg), 0.0)

    dis_src = jnp.take(dis, src, indices_are_sorted=False)
    dst_s, src_s, nrm_raw = lax.sort(
        (dst, src, dis_src * mask_f), num_keys=1, is_stable=False)
    srcs_s = jnp.pad(src_s, (0, EPAD - N_EDGES))
    dsts_s = jnp.pad(dst_s, (0, EPAD - N_EDGES))
    nrm_s = jnp.pad(nrm_raw, (0, EPAD - N_EDGES))
    meta = jnp.stack(
        [
            (dsts_s % ROWS).astype(jnp.float32).reshape(-1, 16),
            nrm_s.reshape(-1, 16),
        ],
        axis=1,
    ).reshape(-1)
    bounds = jnp.searchsorted(
        dst_s, jnp.arange(NT + 1, dtype=jnp.int32) * ROWS
    ).astype(jnp.int32)
    starts = jnp.zeros((NT, 16), jnp.int32)
    starts = starts.at[:, 0].set(bounds[:NT]).at[:, 1].set(bounds[1:])
    starts = starts.reshape(-1)

    x_pad = jnp.pad(x, ((0, NPAD - N_NODES), (0, 0)))
    dis_pad = jnp.pad(dis, (0, NPAD - N_NODES))

    for _ in range(NUM_LAYER):
        x_pad = _propagate(x_pad, srcs_s, meta, starts, dis_pad)
    return x_pad[:N_NODES]


# speedup vs baseline: 3.2439x; 1.6915x over previous
import functools

import jax
import jax.numpy as jnp
from jax import lax
from jax.experimental import pallas as pl
from jax.experimental.pallas import tpu as pltpu
from jax.experimental.pallas import tpu_sc as plsc

N_NODES = 10000
DIM = 256
N_EDGES = 160000
NUM_LAYER = 3

NT = 32
ROWS = 320
NPAD = NT * ROWS
TRASH = ROWS
ACC_ROWS = ROWS + 1
CHUNK = 32
NBUF = 3
EPAD = N_EDGES + 16 * CHUNK
RG = ROWS // 16

_mesh = plsc.VectorSubcoreMesh(core_axis_name="c", subcore_axis_name="s")


def _mo(v, m):
    return pl.multiple_of(v, m)


def _layer_body(xs_hbm, x_hbm, srcs_hbm, meta_hbm, starts_hbm, dis_hbm,
                xout_hbm, xsout_hbm, *scratch):
    sidx = scratch[0:NBUF]
    mb = scratch[NBUF:2 * NBUF]
    rows = scratch[2 * NBUF:3 * NBUF]
    acc_v, xb0, xb1, sb0, sb1, dis_v, meta_v = scratch[3 * NBUF:3 * NBUF + 7]
    sems = scratch[3 * NBUF + 7:]
    msa = sems[0:NBUF]
    msb = sems[NBUF:2 * NBUF]
    gs = sems[2 * NBUF:3 * NBUF]
    xi0, xi1, xo, xs0sem, xs1sem = sems[3 * NBUF:]

    wid = lax.axis_index("c") * 16 + lax.axis_index("s")
    vbase = _mo(wid * ROWS, 8)

    pltpu.sync_copy(starts_hbm.at[pl.ds(_mo(wid * 16, 16), 16)], meta_v)
    mvec = meta_v[...]
    s0 = mvec[0]
    s1 = mvec[1]
    abase = _mo(s0 & ~15, 16)
    nq = (s1 - abase + NBUF * CHUNK - 1) // (NBUF * CHUNK)

    def issue_meta(ci, k):
        eoff = _mo(abase + ci * CHUNK, 16)
        pltpu.async_copy(srcs_hbm.at[pl.ds(eoff, CHUNK)], sidx[k], msa[k])
        pltpu.async_copy(meta_hbm.at[pl.ds(eoff, CHUNK)], mb[k], msb[k])

    def wait_meta(k):
        pltpu.make_async_copy(
            srcs_hbm.at[pl.ds(0, CHUNK)], sidx[k], msa[k]).wait()
        pltpu.make_async_copy(
            meta_hbm.at[pl.ds(0, CHUNK)], mb[k], msb[k]).wait()

    def accumulate(ci, k):
        eoff = abase + ci * CHUNK
        for g in range(CHUNK // 16):
            pos = eoff + g * 16 + lax.iota(jnp.int32, 16)
            valid = (pos >= s0) & (pos < s1)
            dvec = jnp.where(valid, mb[k][pl.ds(16 * g, 16)], TRASH)
            dscal = [dvec[j] for j in range(16)]

            @pl.loop(0, DIM, step=16)
            def _(c):
                cc = _mo(c, 16)
                for j in range(16):
                    e = g * 16 + j
                    plsc.addupdate(
                        acc_v.at[dscal[j], pl.ds(cc, 16)],
                        rows[k][e, pl.ds(cc, 16)],
                    )

    for k in range(NBUF):
        issue_meta(k, k)

    zeros16 = jnp.zeros((16,), jnp.float32)

    @pl.loop(0, ACC_ROWS)
    def _(r):
        for c in range(0, DIM, 16):
            acc_v[r, pl.ds(c, 16)] = zeros16

    @pl.loop(0, nq)
    def _(q):
        base = NBUF * q
        for k in range(NBUF):
            wait_meta(k)
            pltpu.async_copy(xs_hbm.at[sidx[k]], rows[k], gs[k])
        for k in range(NBUF):
            pltpu.make_async_copy(xs_hbm.at[sidx[k]], rows[k], gs[k]).wait()
            accumulate(base + k, k)
            issue_meta(base + k + NBUF, k)

    for k in range(NBUF):
        wait_meta(k)

    pltpu.sync_copy(dis_hbm.at[pl.ds(vbase, ROWS)], dis_v)

    def issue_xin(rg, xb, sem):
        pltpu.async_copy(
            x_hbm.at[pl.ds(vbase + _mo(rg * 16, 16), 16)], xb, sem)

    issue_xin(0, xb0, xi0)
    issue_xin(1, xb1, xi1)

    def epi_step(rg, xb, xsem, sb, ssem):
        rb = _mo(rg * 16, 16)
        pltpu.make_async_copy(x_hbm.at[pl.ds(vbase, 16)], xb, xsem).wait()

        @pl.when(rg >= 2)
        def _():
            pltpu.make_async_copy(
                sb, xsout_hbm.at[pl.ds(vbase, 16)], ssem).wait()

        dvals = dis_v[pl.ds(rb, 16)]
        dscal = [dvals[j] for j in range(16)]

        @pl.loop(0, DIM, step=16)
        def _(c):
            cc = _mo(c, 16)
            for j in range(16):
                t = xb[j, pl.ds(cc, 16)] + dscal[j] * acc_v[rb + j, pl.ds(cc, 16)]
                acc_v[rb + j, pl.ds(cc, 16)] = t
                sb[j, pl.ds(cc, 16)] = dscal[j] * t

        pltpu.async_copy(acc_v.at[pl.ds(rb, 16)],
                         xout_hbm.at[pl.ds(vbase + rb, 16)], xo)
        pltpu.async_copy(sb, xsout_hbm.at[pl.ds(vbase + rb, 16)], ssem)

        @pl.when(rg + 2 < RG)
        def _():
            issue_xin(rg + 2, xb, xsem)

    @pl.loop(0, RG // 2)
    def _(q):
        epi_step(2 * q, xb0, xi0, sb0, xs0sem)
        epi_step(2 * q + 1, xb1, xi1, sb1, xs1sem)

    @pl.loop(0, RG)
    def _(r):
        pltpu.make_async_copy(acc_v.at[pl.ds(0, 16)],
                              xout_hbm.at[pl.ds(0, 16)], xo).wait()
    pltpu.make_async_copy(sb0, xsout_hbm.at[pl.ds(vbase, 16)], xs0sem).wait()
    pltpu.make_async_copy(sb1, xsout_hbm.at[pl.ds(vbase, 16)], xs1sem).wait()


_sds = jax.ShapeDtypeStruct((NPAD, DIM), jnp.float32)

_propagate = functools.partial(
    pl.kernel,
    out_type=(_sds, _sds),
    mesh=_mesh,
    scratch_types=(
        [pltpu.VMEM((CHUNK,), jnp.int32) for _ in range(NBUF)]
        + [pltpu.VMEM((CHUNK,), jnp.int32) for _ in range(NBUF)]
        + [pltpu.VMEM((CHUNK, DIM), jnp.float32) for _ in range(NBUF)]
        + [
            pltpu.VMEM((ACC_ROWS, DIM), jnp.float32),
            pltpu.VMEM((16, DIM), jnp.float32),
            pltpu.VMEM((16, DIM), jnp.float32),
            pltpu.VMEM((16, DIM), jnp.float32),
            pltpu.VMEM((16, DIM), jnp.float32),
            pltpu.VMEM((ROWS,), jnp.float32),
            pltpu.VMEM((16,), jnp.int32),
        ]
        + [pltpu.SemaphoreType.DMA for _ in range(3 * NBUF + 5)]
    ),
)(_layer_body)


def kernel(edge_index, user, item):
    src = edge_index[0].astype(jnp.int32)
    dst = edge_index[1].astype(jnp.int32)
    x = jnp.concatenate([user, item], axis=0)

    mask_f = (src != dst).astype(jnp.float32)
    deg = jnp.zeros((N_NODES,), jnp.float32).at[src].add(mask_f)
    dis = jnp.where(deg > 0, lax.rsqrt(deg), 0.0)

    dst_s, src_s = lax.sort((dst, src), num_keys=1, is_stable=False)
    dloc = jnp.where(src_s == dst_s, TRASH, dst_s % ROWS)
    srcs_s = jnp.pad(src_s, (0, EPAD - N_EDGES))
    meta = jnp.pad(dloc, (0, EPAD - N_EDGES), constant_values=TRASH)
    bounds = jnp.searchsorted(
        dst_s, jnp.arange(NT + 1, dtype=jnp.int32) * ROWS
    ).astype(jnp.int32)
    starts = jnp.zeros((NT, 16), jnp.int32)
    starts = starts.at[:, 0].set(bounds[:NT]).at[:, 1].set(bounds[1:])
    starts = starts.reshape(-1)

    x_pad = jnp.pad(x, ((0, NPAD - N_NODES), (0, 0)))
    dis_pad = jnp.pad(dis, (0, NPAD - N_NODES))
    xs_pad = dis_pad[:, None] * x_pad

    for _ in range(NUM_LAYER):
        x_pad, xs_pad = _propagate(xs_pad, x_pad, srcs_s, meta, starts,
                                   dis_pad)
    return x_pad[:N_NODES]
